# baseline (speedup 1.0000x reference)
import jax
import jax.numpy as jnp
from jax import lax
from jax.experimental import pallas as pl
from jax.experimental.pallas import tpu as pltpu
from jax.experimental.pallas import tpu_sc as plsc

N = 10000
E = 160000
F_IN = 256
H = 512

NC = 2
NS = 16
NW = NC * NS
L = 16

NB = 64
RPB = 157
NPAD = NB * RPB
CAP = 4096
CH = 8000
G = 32

_mesh = plsc.VectorSubcoreMesh(core_axis_name="c", subcore_axis_name="s")
_sc_params = pltpu.CompilerParams(needs_layout_passes=False)


def _prep_body(src_hbm, dst_hbm, ew_hbm,
               srcb_hbm, dlb_hbm, ewb_hbm, cnt_hbm, deg_hbm,
               src_c, dst_c, ew_c,
               src0, dl0, ew0, src1, dl1, ew1,
               deg0, deg1, cnt_l):
    wid = lax.axis_index("s") * NC + lax.axis_index("c")
    b0 = 2 * wid
    b1 = b0 + 1
    lo0 = b0 * RPB
    lo1 = b1 * RPB

    zi = jnp.zeros((L,), jnp.int32)
    zf = jnp.zeros((L,), jnp.float32)
    iota = lax.iota(jnp.int32, L)

    def zero_buckets(i, _):
        sl = pl.ds(i * L, L)
        src0[sl] = zi
        dl0[sl] = zi
        ew0[sl] = zf
        src1[sl] = zi
        dl1[sl] = zi
        ew1[sl] = zf
        return 0
    lax.fori_loop(0, (CAP + L) // L, zero_buckets, 0)

    def zero_deg(i, _):
        sl = pl.ds(i * L, L)
        deg0[sl] = zf
        deg1[sl] = zf
        return 0
    lax.fori_loop(0, 160 // L, zero_deg, 0)

    def chunk_body(c, ptrs):
        off = c * CH
        pltpu.sync_copy(src_hbm.at[pl.ds(off, CH)], src_c)
        pltpu.sync_copy(dst_hbm.at[pl.ds(off, CH)], dst_c)
        pltpu.sync_copy(ew_hbm.at[pl.ds(off, CH)], ew_c)

        def vec_body(j, ptrs):
            p0, p1 = ptrs
            sl = pl.ds(j * L, L)
            s = src_c[sl]
            dv = dst_c[sl]
            w = ew_c[sl]
            m0 = (dv >= lo0) & (dv < lo0 + RPB)
            m1 = (dv >= lo1) & (dv < lo1 + RPB)
            i0 = m0.astype(jnp.int32)
            i1 = m1.astype(jnp.int32)
            pos0 = jnp.where(m0, p0 + jnp.cumsum(i0) - 1, CAP + iota)
            pos1 = jnp.where(m1, p1 + jnp.cumsum(i1) - 1, CAP + iota)
            plsc.store_scatter(src0, [pos0], s)
            plsc.store_scatter(dl0, [pos0], dv - lo0)
            plsc.store_scatter(ew0, [pos0], w)
            plsc.store_scatter(src1, [pos1], s)
            plsc.store_scatter(dl1, [pos1], dv - lo1)
            plsc.store_scatter(ew1, [pos1], w)
            p0 = jnp.minimum(p0 + jnp.sum(i0), CAP - L)
            p1 = jnp.minimum(p1 + jnp.sum(i1), CAP - L)
            return (p0, p1)

        return lax.fori_loop(0, CH // L, vec_body, ptrs)

    p0, p1 = lax.fori_loop(0, E // CH, chunk_body,
                           (jnp.int32(0), jnp.int32(0)))

    def deg_add(i, _):
        sl = pl.ds(i * L, L)
        plsc.addupdate_scatter(deg0, [dl0[sl]], ew0[sl])
        plsc.addupdate_scatter(deg1, [dl1[sl]], ew1[sl])
        return 0
    lax.fori_loop(0, CAP // L, deg_add, 0)

    one = jnp.full((L,), 1.0, jnp.float32)

    def deg_selfloop(i, _):
        sl = pl.ds(i * L, L)
        plsc.addupdate(deg0.at[sl], one)
        plsc.addupdate(deg1.at[sl], one)
        return 0
    lax.fori_loop(0, 160 // L, deg_selfloop, 0)

    pltpu.sync_copy(src0.at[pl.ds(0, CAP)], srcb_hbm.at[b0])
    pltpu.sync_copy(dl0.at[pl.ds(0, CAP)], dlb_hbm.at[b0])
    pltpu.sync_copy(ew0.at[pl.ds(0, CAP)], ewb_hbm.at[b0])
    pltpu.sync_copy(src1.at[pl.ds(0, CAP)], srcb_hbm.at[b1])
    pltpu.sync_copy(dl1.at[pl.ds(0, CAP)], dlb_hbm.at[b1])
    pltpu.sync_copy(ew1.at[pl.ds(0, CAP)], ewb_hbm.at[b1])
    pltpu.sync_copy(deg0, deg_hbm.at[b0])
    pltpu.sync_copy(deg1, deg_hbm.at[b1])
    cnt_l[pl.ds(0, L)] = zi + p0
    pltpu.sync_copy(cnt_l, cnt_hbm.at[b0])
    cnt_l[pl.ds(0, L)] = zi + p1
    pltpu.sync_copy(cnt_l, cnt_hbm.at[b1])


_prep_call = pl.kernel(
    _prep_body,
    out_type=[
        jax.ShapeDtypeStruct((NB, CAP), jnp.int32),
        jax.ShapeDtypeStruct((NB, CAP), jnp.int32),
        jax.ShapeDtypeStruct((NB, CAP), jnp.float32),
        jax.ShapeDtypeStruct((NB, L), jnp.int32),
        jax.ShapeDtypeStruct((NB, 160), jnp.float32),
    ],
    mesh=_mesh,
    scratch_types=[
        pltpu.VMEM((CH,), jnp.int32),
        pltpu.VMEM((CH,), jnp.int32),
        pltpu.VMEM((CH,), jnp.float32),
        pltpu.VMEM((CAP + L,), jnp.int32),
        pltpu.VMEM((CAP + L,), jnp.int32),
        pltpu.VMEM((CAP + L,), jnp.float32),
        pltpu.VMEM((CAP + L,), jnp.int32),
        pltpu.VMEM((CAP + L,), jnp.int32),
        pltpu.VMEM((CAP + L,), jnp.float32),
        pltpu.VMEM((160,), jnp.float32),
        pltpu.VMEM((160,), jnp.float32),
        pltpu.VMEM((L,), jnp.int32),
    ],
    compiler_params=_sc_params,
)


def _lane_splat(vec, t):
    idx = jnp.full((L, 1), t, jnp.int32)
    dn = lax.GatherDimensionNumbers(
        offset_dims=(), collapsed_slice_dims=(0,), start_index_map=(0,))
    return lax.gather(vec, idx, dn, (1,),
                      mode=lax.GatherScatterMode.PROMISE_IN_BOUNDS)


def _conv_body(y_hbm, srcb_hbm, dlb_hbm, ewb_hbm, cnt_hbm,
               z_hbm,
               src_l, dl_l, ew_l, cnt_l, stag0, stag1, out_l, sem0, sem1):
    wid = lax.axis_index("s") * NC + lax.axis_index("c")
    zf = jnp.zeros((L,), jnp.float32)
    iota = lax.iota(jnp.int32, L)

    def process(base, stag):
        def jj_body(jj, _):
            off = base + jj * L
            ew_vec = ew_l[pl.ds(off, L)]
            dl_vec = dl_l[pl.ds(off, L)]
            for t in range(L):
                wj = _lane_splat(ew_vec, t)
                row_base = _lane_splat(dl_vec, t) * H
                for k in range(H // L):
                    idx = row_base + (k * L) + iota
                    plsc.addupdate_scatter(
                        out_l, [idx],
                        wj * stag[jj * L + t, pl.ds(k * L, L)])
            return 0
        lax.fori_loop(0, G // L, jj_body, 0)

    def round_body(r, _):
        b = 2 * wid + r
        lo = b * RPB

        def zero_vec(k, _):
            out_l[pl.ds(k * L, L)] = zf
            return 0
        lax.fori_loop(0, RPB * H // L, zero_vec, 0)

        pltpu.sync_copy(srcb_hbm.at[b], src_l)
        pltpu.sync_copy(dlb_hbm.at[b], dl_l)
        pltpu.sync_copy(ewb_hbm.at[b], ew_l)
        pltpu.sync_copy(cnt_hbm.at[b], cnt_l)
        count = jnp.max(cnt_l[pl.ds(0, L)])
        npair = (count + (2 * G - 1)) >> 6

        def pair_body(i, _):
            base = i * (2 * G)
            cp0 = pltpu.async_copy(
                y_hbm.at[src_l.at[pl.ds(base, G)]], stag0, sem0)
            cp1 = pltpu.async_copy(
                y_hbm.at[src_l.at[pl.ds(base + G, G)]], stag1, sem1)
            cp0.wait()
            process(base, stag0)
            cp1.wait()
            process(base + G, stag1)
            return 0
        lax.fori_loop(0, npair, pair_body, 0)

        pltpu.sync_copy(out_l, z_hbm.at[pl.ds(lo * H, RPB * H)])
        return 0
    lax.fori_loop(0, 2, round_body, 0)


_conv_call = pl.kernel(
    _conv_body,
    out_type=jax.ShapeDtypeStruct((NPAD * H,), jnp.float32),
    mesh=_mesh,
    scratch_types=[
        pltpu.VMEM((CAP,), jnp.int32),
        pltpu.VMEM((CAP,), jnp.int32),
        pltpu.VMEM((CAP,), jnp.float32),
        pltpu.VMEM((L,), jnp.int32),
        pltpu.VMEM((G, H), jnp.float32),
        pltpu.VMEM((G, H), jnp.float32),
        pltpu.VMEM((RPB * H,), jnp.float32),
        pltpu.SemaphoreType.DMA,
        pltpu.SemaphoreType.DMA,
    ],
    compiler_params=_sc_params,
)


def _ew_body(ea_ref, we_ref, be_ref, o_ref):
    w = we_ref[...].reshape(1, 8)
    s = jnp.sum(ea_ref[...] * w, axis=1, keepdims=True) + be_ref[...]
    o_ref[...] = jax.nn.sigmoid(s)


def _edge_weights(edge_attr, We, be):
    blk = E // 8
    return pl.pallas_call(
        _ew_body,
        grid=(8,),
        in_specs=[
            pl.BlockSpec((blk, 8), lambda i: (i, 0)),
            pl.BlockSpec((8, 1), lambda i: (0, 0)),
            pl.BlockSpec((1, 1), lambda i: (0, 0)),
        ],
        out_specs=pl.BlockSpec((blk, 1), lambda i: (i, 0)),
        out_shape=jax.ShapeDtypeStruct((E, 1), jnp.float32),
    )(edge_attr, We, be.reshape(1, 1))


def _mm1_body(x_ref, w_ref, deg_ref, o_ref):
    dis = lax.rsqrt(deg_ref[...])
    o_ref[...] = dis * jnp.dot(x_ref[...], w_ref[...],
                               preferred_element_type=jnp.float32)


def _scaled_matmul(x, W, degcol):
    blk = 1000
    f_in = x.shape[1]
    return pl.pallas_call(
        _mm1_body,
        grid=(N // blk,),
        in_specs=[
            pl.BlockSpec((blk, f_in), lambda i: (i, 0)),
            pl.BlockSpec((f_in, H), lambda i: (0, 0)),
            pl.BlockSpec((blk, 1), lambda i: (i, 0)),
        ],
        out_specs=pl.BlockSpec((blk, H), lambda i: (i, 0)),
        out_shape=jax.ShapeDtypeStruct((N, H), jnp.float32),
    )(x, W, degcol)


def _mid_body(z_ref, y_ref, deg_ref, b_ref, w_ref, o_ref):
    dis = lax.rsqrt(deg_ref[...])
    h = jnp.maximum(dis * (z_ref[...] + y_ref[...]) + b_ref[...], 0.0)
    o_ref[...] = dis * jnp.dot(h, w_ref[...],
                               preferred_element_type=jnp.float32)


def _mid_layer(z, y, degcol, b, W):
    blk = 1000
    return pl.pallas_call(
        _mid_body,
        grid=(N // blk,),
        in_specs=[
            pl.BlockSpec((blk, H), lambda i: (i, 0)),
            pl.BlockSpec((blk, H), lambda i: (i, 0)),
            pl.BlockSpec((blk, 1), lambda i: (i, 0)),
            pl.BlockSpec((1, H), lambda i: (0, 0)),
            pl.BlockSpec((H, H), lambda i: (0, 0)),
        ],
        out_specs=pl.BlockSpec((blk, H), lambda i: (i, 0)),
        out_shape=jax.ShapeDtypeStruct((N, H), jnp.float32),
    )(z, y, degcol, b.reshape(1, H), W)


def _out_body(z_ref, y_ref, deg_ref, b_ref, wo_ref, bo_ref, o_ref):
    dis = lax.rsqrt(deg_ref[...])
    h = jnp.maximum(dis * (z_ref[...] + y_ref[...]) + b_ref[...], 0.0)
    o_ref[...] = jnp.dot(h, wo_ref[...],
                         preferred_element_type=jnp.float32) + bo_ref[...]


def _out_layer(z, y, degcol, b, Wout, bout):
    blk = 1000
    return pl.pallas_call(
        _out_body,
        grid=(N // blk,),
        in_specs=[
            pl.BlockSpec((blk, H), lambda i: (i, 0)),
            pl.BlockSpec((blk, H), lambda i: (i, 0)),
            pl.BlockSpec((blk, 1), lambda i: (i, 0)),
            pl.BlockSpec((1, H), lambda i: (0, 0)),
            pl.BlockSpec((H, 1), lambda i: (0, 0)),
            pl.BlockSpec((1, 1), lambda i: (0, 0)),
        ],
        out_specs=pl.BlockSpec((blk, 1), lambda i: (i, 0)),
        out_shape=jax.ShapeDtypeStruct((N, 1), jnp.float32),
    )(z, y, degcol, b.reshape(1, H), Wout, bout.reshape(1, 1))


def kernel(x, edge_index, edge_attr, We, be, W1, b1, W2, b2, Wout, bout):
    src = edge_index[0]
    dst = edge_index[1]

    ew = _edge_weights(edge_attr, We, be).reshape(E)
    srcb, dlb, ewb, cnt, deg2d = _prep_call(src, dst, ew)
    degcol = deg2d[:, :RPB].reshape(NPAD)[:N].reshape(N, 1)

    y1 = _scaled_matmul(x, W1, degcol)
    z1 = _conv_call(y1, srcb, dlb, ewb, cnt).reshape(NPAD, H)[:N]
    y2 = _mid_layer(z1, y1, degcol, b1, W2)
    z2 = _conv_call(y2, srcb, dlb, ewb, cnt).reshape(NPAD, H)[:N]
    return _out_layer(z2, y2, degcol, b2, Wout, bout)


# speedup vs baseline: 1.2981x; 1.2981x over previous
import jax
import jax.numpy as jnp
from jax import lax
from jax.experimental import pallas as pl
from jax.experimental.pallas import tpu as pltpu
from jax.experimental.pallas import tpu_sc as plsc

N = 10000
E = 160000
F_IN = 256
H = 512

NC = 2
NS = 16
NW = NC * NS
L = 16

NB = 64
RPB = 157
NPAD = NB * RPB
CAP = 4096
CH = 8000
G = 32

_mesh = plsc.VectorSubcoreMesh(core_axis_name="c", subcore_axis_name="s")
_sc_params = pltpu.CompilerParams(needs_layout_passes=False)


def _prep_body(src_hbm, dst_hbm, ew_hbm,
               srcb_hbm, dlb_hbm, ewb_hbm, cnt_hbm, deg_hbm,
               src_c, dst_c, ew_c,
               src0, dl0, ew0, src1, dl1, ew1,
               deg0, deg1, cnt_l):
    wid = lax.axis_index("s") * NC + lax.axis_index("c")
    b0 = 2 * wid
    b1 = b0 + 1
    lo0 = b0 * RPB
    lo1 = b1 * RPB

    zi = jnp.zeros((L,), jnp.int32)
    zf = jnp.zeros((L,), jnp.float32)
    iota = lax.iota(jnp.int32, L)

    def zero_buckets(i, _):
        sl = pl.ds(i * L, L)
        src0[sl] = zi
        dl0[sl] = zi
        ew0[sl] = zf
        src1[sl] = zi
        dl1[sl] = zi
        ew1[sl] = zf
        return 0
    lax.fori_loop(0, (CAP + L) // L, zero_buckets, 0)

    def zero_deg(i, _):
        sl = pl.ds(i * L, L)
        deg0[sl] = zf
        deg1[sl] = zf
        return 0
    lax.fori_loop(0, 160 // L, zero_deg, 0)

    def chunk_body(c, ptrs):
        off = c * CH
        pltpu.sync_copy(src_hbm.at[pl.ds(off, CH)], src_c)
        pltpu.sync_copy(dst_hbm.at[pl.ds(off, CH)], dst_c)
        pltpu.sync_copy(ew_hbm.at[pl.ds(off, CH)], ew_c)

        def vec_body(j, ptrs):
            p0, p1 = ptrs
            sl = pl.ds(j * L, L)
            s = src_c[sl]
            dv = dst_c[sl]
            w = ew_c[sl]
            m0 = (dv >= lo0) & (dv < lo0 + RPB)
            m1 = (dv >= lo1) & (dv < lo1 + RPB)
            i0 = m0.astype(jnp.int32)
            i1 = m1.astype(jnp.int32)
            pos0 = jnp.where(m0, p0 + jnp.cumsum(i0) - 1, CAP + iota)
            pos1 = jnp.where(m1, p1 + jnp.cumsum(i1) - 1, CAP + iota)
            plsc.store_scatter(src0, [pos0], s)
            plsc.store_scatter(dl0, [pos0], dv - lo0)
            plsc.store_scatter(ew0, [pos0], w)
            plsc.store_scatter(src1, [pos1], s)
            plsc.store_scatter(dl1, [pos1], dv - lo1)
            plsc.store_scatter(ew1, [pos1], w)
            p0 = jnp.minimum(p0 + jnp.sum(i0), CAP - L)
            p1 = jnp.minimum(p1 + jnp.sum(i1), CAP - L)
            return (p0, p1)

        return lax.fori_loop(0, CH // L, vec_body, ptrs)

    p0, p1 = lax.fori_loop(0, E // CH, chunk_body,
                           (jnp.int32(0), jnp.int32(0)))

    def deg_add(i, _):
        sl = pl.ds(i * L, L)
        plsc.addupdate_scatter(deg0, [dl0[sl]], ew0[sl])
        plsc.addupdate_scatter(deg1, [dl1[sl]], ew1[sl])
        return 0
    lax.fori_loop(0, CAP // L, deg_add, 0)

    one = jnp.full((L,), 1.0, jnp.float32)

    def deg_selfloop(i, _):
        sl = pl.ds(i * L, L)
        plsc.addupdate(deg0.at[sl], one)
        plsc.addupdate(deg1.at[sl], one)
        return 0
    lax.fori_loop(0, 160 // L, deg_selfloop, 0)

    pltpu.sync_copy(src0.at[pl.ds(0, CAP)], srcb_hbm.at[b0])
    pltpu.sync_copy(dl0.at[pl.ds(0, CAP)], dlb_hbm.at[b0])
    pltpu.sync_copy(ew0.at[pl.ds(0, CAP)], ewb_hbm.at[b0])
    pltpu.sync_copy(src1.at[pl.ds(0, CAP)], srcb_hbm.at[b1])
    pltpu.sync_copy(dl1.at[pl.ds(0, CAP)], dlb_hbm.at[b1])
    pltpu.sync_copy(ew1.at[pl.ds(0, CAP)], ewb_hbm.at[b1])
    pltpu.sync_copy(deg0, deg_hbm.at[b0])
    pltpu.sync_copy(deg1, deg_hbm.at[b1])
    cnt_l[pl.ds(0, L)] = zi + p0
    pltpu.sync_copy(cnt_l, cnt_hbm.at[b0])
    cnt_l[pl.ds(0, L)] = zi + p1
    pltpu.sync_copy(cnt_l, cnt_hbm.at[b1])


_prep_call = pl.kernel(
    _prep_body,
    out_type=[
        jax.ShapeDtypeStruct((NB, CAP), jnp.int32),
        jax.ShapeDtypeStruct((NB, CAP), jnp.int32),
        jax.ShapeDtypeStruct((NB, CAP), jnp.float32),
        jax.ShapeDtypeStruct((NB, L), jnp.int32),
        jax.ShapeDtypeStruct((NB, 160), jnp.float32),
    ],
    mesh=_mesh,
    scratch_types=[
        pltpu.VMEM((CH,), jnp.int32),
        pltpu.VMEM((CH,), jnp.int32),
        pltpu.VMEM((CH,), jnp.float32),
        pltpu.VMEM((CAP + L,), jnp.int32),
        pltpu.VMEM((CAP + L,), jnp.int32),
        pltpu.VMEM((CAP + L,), jnp.float32),
        pltpu.VMEM((CAP + L,), jnp.int32),
        pltpu.VMEM((CAP + L,), jnp.int32),
        pltpu.VMEM((CAP + L,), jnp.float32),
        pltpu.VMEM((160,), jnp.float32),
        pltpu.VMEM((160,), jnp.float32),
        pltpu.VMEM((L,), jnp.int32),
    ],
    compiler_params=_sc_params,
)


def _lane_splat(vec, t):
    idx = jnp.full((L, 1), t, jnp.int32)
    dn = lax.GatherDimensionNumbers(
        offset_dims=(), collapsed_slice_dims=(0,), start_index_map=(0,))
    return lax.gather(vec, idx, dn, (1,),
                      mode=lax.GatherScatterMode.PROMISE_IN_BOUNDS)


def _conv_body(y_hbm, srcb_hbm, dlb_hbm, ewb_hbm, cnt_hbm,
               z_hbm,
               src_l, dl_l, ew_l, cnt_l, stag0, stag1, out_l, sem0, sem1):
    wid = lax.axis_index("s") * NC + lax.axis_index("c")
    zf = jnp.zeros((L,), jnp.float32)
    zi = jnp.zeros((L,), jnp.int32)
    iota = lax.iota(jnp.int32, L)

    def round_body(r, _):
        b = 2 * wid + r
        lo = b * RPB

        def zero_vec(k, _):
            out_l[pl.ds(k * L, L)] = zf
            return 0
        lax.fori_loop(0, RPB * H // L, zero_vec, 0)

        pltpu.sync_copy(srcb_hbm.at[b], src_l.at[pl.ds(0, CAP)])
        pltpu.sync_copy(dlb_hbm.at[b], dl_l)
        pltpu.sync_copy(ewb_hbm.at[b], ew_l)
        pltpu.sync_copy(cnt_hbm.at[b], cnt_l)
        for q in range(G // L):
            src_l[pl.ds(CAP + q * L, L)] = zi
        count = jnp.max(cnt_l[pl.ds(0, L)])
        npairs = (count + (2 * G - 1)) >> 6

        def process(stag_b, base):
            for jj in range(G // L):
                sl = pl.ds(base + jj * L, L)
                ew_vec = ew_l[sl]
                rb_vec = dl_l[sl] * H
                for t in range(L):
                    j_row = jj * L + t
                    wj = _lane_splat(ew_vec, t)
                    rb = jnp.max(jnp.where(iota == t, rb_vec, 0))
                    for k in range(H // L):
                        plsc.addupdate(
                            out_l.at[pl.ds(rb + k * L, L)],
                            wj * stag_b[j_row, pl.ds(k * L, L)])

        pltpu.async_copy(y_hbm.at[src_l.at[pl.ds(0, G)]], stag0, sem0)

        def pair_body(p, _):
            base0 = (2 * p) * G
            pltpu.async_copy(
                y_hbm.at[src_l.at[pl.ds(base0 + G, G)]], stag1, sem1)
            pltpu.make_async_copy(
                y_hbm.at[src_l.at[pl.ds(0, G)]], stag0, sem0).wait()
            process(stag0, base0)
            pltpu.async_copy(
                y_hbm.at[src_l.at[pl.ds(base0 + 2 * G, G)]], stag0, sem0)
            pltpu.make_async_copy(
                y_hbm.at[src_l.at[pl.ds(0, G)]], stag1, sem1).wait()
            process(stag1, base0 + G)
            return 0
        lax.fori_loop(0, npairs, pair_body, 0)

        pltpu.make_async_copy(
            y_hbm.at[src_l.at[pl.ds(0, G)]], stag0, sem0).wait()

        pltpu.sync_copy(out_l, z_hbm.at[pl.ds(lo * H, RPB * H)])
        return 0
    lax.fori_loop(0, 2, round_body, 0)


_conv_call = pl.kernel(
    _conv_body,
    out_type=jax.ShapeDtypeStruct((NPAD * H,), jnp.float32),
    mesh=_mesh,
    scratch_types=[
        pltpu.VMEM((CAP + G,), jnp.int32),
        pltpu.VMEM((CAP,), jnp.int32),
        pltpu.VMEM((CAP,), jnp.float32),
        pltpu.VMEM((L,), jnp.int32),
        pltpu.VMEM((G, H), jnp.float32),
        pltpu.VMEM((G, H), jnp.float32),
        pltpu.VMEM((RPB * H,), jnp.float32),
        pltpu.SemaphoreType.DMA,
        pltpu.SemaphoreType.DMA,
    ],
    compiler_params=_sc_params,
)


def _ew_body(ea_ref, we_ref, be_ref, o_ref):
    w = we_ref[...].reshape(1, 8)
    s = jnp.sum(ea_ref[...] * w, axis=1, keepdims=True) + be_ref[...]
    o_ref[...] = jax.nn.sigmoid(s)


def _edge_weights(edge_attr, We, be):
    blk = E // 8
    return pl.pallas_call(
        _ew_body,
        grid=(8,),
        in_specs=[
            pl.BlockSpec((blk, 8), lambda i: (i, 0)),
            pl.BlockSpec((8, 1), lambda i: (0, 0)),
            pl.BlockSpec((1, 1), lambda i: (0, 0)),
        ],
        out_specs=pl.BlockSpec((blk, 1), lambda i: (i, 0)),
        out_shape=jax.ShapeDtypeStruct((E, 1), jnp.float32),
    )(edge_attr, We, be.reshape(1, 1))


def _mm1_body(x_ref, w_ref, deg_ref, o_ref):
    dis = lax.rsqrt(deg_ref[...])
    o_ref[...] = dis * jnp.dot(x_ref[...], w_ref[...],
                               preferred_element_type=jnp.float32)


def _scaled_matmul(x, W, degcol):
    blk = 1000
    f_in = x.shape[1]
    return pl.pallas_call(
        _mm1_body,
        grid=(N // blk,),
        in_specs=[
            pl.BlockSpec((blk, f_in), lambda i: (i, 0)),
            pl.BlockSpec((f_in, H), lambda i: (0, 0)),
            pl.BlockSpec((blk, 1), lambda i: (i, 0)),
        ],
        out_specs=pl.BlockSpec((blk, H), lambda i: (i, 0)),
        out_shape=jax.ShapeDtypeStruct((N, H), jnp.float32),
    )(x, W, degcol)


def _mid_body(z_ref, y_ref, deg_ref, b_ref, w_ref, o_ref):
    dis = lax.rsqrt(deg_ref[...])
    h = jnp.maximum(dis * (z_ref[...] + y_ref[...]) + b_ref[...], 0.0)
    o_ref[...] = dis * jnp.dot(h, w_ref[...],
                               preferred_element_type=jnp.float32)


def _mid_layer(z, y, degcol, b, W):
    blk = 1000
    return pl.pallas_call(
        _mid_body,
        grid=(N // blk,),
        in_specs=[
            pl.BlockSpec((blk, H), lambda i: (i, 0)),
            pl.BlockSpec((blk, H), lambda i: (i, 0)),
            pl.BlockSpec((blk, 1), lambda i: (i, 0)),
            pl.BlockSpec((1, H), lambda i: (0, 0)),
            pl.BlockSpec((H, H), lambda i: (0, 0)),
        ],
        out_specs=pl.BlockSpec((blk, H), lambda i: (i, 0)),
        out_shape=jax.ShapeDtypeStruct((N, H), jnp.float32),
    )(z, y, degcol, b.reshape(1, H), W)


def _out_body(z_ref, y_ref, deg_ref, b_ref, wo_ref, bo_ref, o_ref):
    dis = lax.rsqrt(deg_ref[...])
    h = jnp.maximum(dis * (z_ref[...] + y_ref[...]) + b_ref[...], 0.0)
    o_ref[...] = jnp.dot(h, wo_ref[...],
                         preferred_element_type=jnp.float32) + bo_ref[...]


def _out_layer(z, y, degcol, b, Wout, bout):
    blk = 1000
    return pl.pallas_call(
        _out_body,
        grid=(N // blk,),
        in_specs=[
            pl.BlockSpec((blk, H), lambda i: (i, 0)),
            pl.BlockSpec((blk, H), lambda i: (i, 0)),
            pl.BlockSpec((blk, 1), lambda i: (i, 0)),
            pl.BlockSpec((1, H), lambda i: (0, 0)),
            pl.BlockSpec((H, 1), lambda i: (0, 0)),
            pl.BlockSpec((1, 1), lambda i: (0, 0)),
        ],
        out_specs=pl.BlockSpec((blk, 1), lambda i: (i, 0)),
        out_shape=jax.ShapeDtypeStruct((N, 1), jnp.float32),
    )(z, y, degcol, b.reshape(1, H), Wout, bout.reshape(1, 1))


def kernel(x, edge_index, edge_attr, We, be, W1, b1, W2, b2, Wout, bout):
    src = edge_index[0]
    dst = edge_index[1]

    ew = _edge_weights(edge_attr, We, be).reshape(E)
    srcb, dlb, ewb, cnt, deg2d = _prep_call(src, dst, ew)
    degcol = deg2d[:, :RPB].reshape(NPAD)[:N].reshape(N, 1)

    y1 = _scaled_matmul(x, W1, degcol)
    z1 = _conv_call(y1, srcb, dlb, ewb, cnt).reshape(NPAD, H)[:N]
    y2 = _mid_layer(z1, y1, degcol, b1, W2)
    z2 = _conv_call(y2, srcb, dlb, ewb, cnt).reshape(NPAD, H)[:N]
    return _out_layer(z2, y2, degcol, b2, Wout, bout)
